# Initial kernel scaffold; baseline (speedup 1.0000x reference)
import jax
import jax.numpy as jnp
from jax.experimental import pallas as pl


def kernel(q, k, v):
    raise NotImplementedError("write your pallas kernel here")


# speedup vs baseline: 22.1629x; 22.1629x over previous
import functools

import jax
import jax.numpy as jnp
from jax.experimental import pallas as pl
from jax.experimental.pallas import tpu as pltpu

TILE_N = 256
KNN = 3


def _tc_body(qT_ref, k_ref, vT_ref, out_ref):
    qT = qT_ref[0]
    k = k_ref[0]
    S = k.shape[0]
    T = qT.shape[1]

    qx = qT[0:1, :]
    qy = qT[1:2, :]
    qz = qT[2:3, :]
    kx = k[:, 0:1]
    ky = k[:, 1:2]
    kz = k[:, 2:3]

    q2 = qx * qx + qy * qy + qz * qz
    k2 = kx * kx + ky * ky + kz * kz
    qk = jnp.dot(k.astype(jnp.bfloat16), qT.astype(jnp.bfloat16),
                 preferred_element_type=jnp.float32)
    dist = q2 + k2 - 2.0 * qk

    iota = jax.lax.broadcasted_iota(jnp.int32, (S, T), 0)
    big = jnp.float32(jnp.inf)

    d = dist
    vals = []
    idxs = []
    for _ in range(KNN):
        m = jnp.min(d, axis=0, keepdims=True)
        cand = jnp.where(d == m, iota, S)
        ix = jnp.min(cand, axis=0, keepdims=True)
        vals.append(m)
        idxs.append(ix)
        d = jnp.where(iota == ix, big, d)

    recips = [1.0 / (m + 1e-8) for m in vals]
    norm = recips[0] + recips[1] + recips[2]

    scoreT = jnp.zeros((S, T), jnp.float32)
    for m_r, ix in zip(recips, idxs):
        w = m_r / norm
        scoreT = scoreT + jnp.where(iota == ix, w, 0.0)

    out_ref[0] = jnp.dot(vT_ref[0], scoreT,
                         preferred_element_type=jnp.float32)


def kernel(q, k, v):
    B, N, _ = q.shape
    S = k.shape[1]
    C = v.shape[2]
    qT = jnp.swapaxes(q, 1, 2)
    vT = jnp.swapaxes(v, 1, 2)

    grid = (B, N // TILE_N)
    out = pl.pallas_call(
        _tc_body,
        grid=grid,
        in_specs=[
            pl.BlockSpec((1, 3, TILE_N), lambda b, i: (b, 0, i)),
            pl.BlockSpec((1, S, 3), lambda b, i: (b, 0, 0)),
            pl.BlockSpec((1, C, S), lambda b, i: (b, 0, 0)),
        ],
        out_specs=pl.BlockSpec((1, C, TILE_N), lambda b, i: (b, 0, i)),
        out_shape=jax.ShapeDtypeStruct((B, C, N), jnp.float32),
    )(qT, k, vT)
    return out
